# baseline (speedup 1.0000x reference)
import jax
import jax.numpy as jnp
import numpy as np
from jax import lax
from jax.experimental import pallas as pl
from jax.experimental.pallas import tpu as pltpu
from jax.experimental.pallas import tpu_sc as plsc

K = 256
IGNORED = -3000.0
PAD_VAL = -1e38

BATCH = 128
VOCAB = 100000
VP = VOCAB
NV = VP // 16
NWORK = 32
ROWS_PER_W = BATCH // NWORK
STAG = 304
SIGN = np.uint32(0x80000000)
ALLON = np.uint32(0xFFFFFFFF)


def _key_of(x):
    b = plsc.bitcast(x, jnp.uint32)
    neg = b >= SIGN
    return b ^ jnp.where(neg, ALLON, SIGN)


def _scalar(v):
    return jnp.max(v)


def _sc_body(logits, out_v, out_i, row_v, hist_v, stag_v, stag_i):
    wid = lax.axis_index("s") * 2 + lax.axis_index("c")
    lane = lax.broadcasted_iota(jnp.int32, (16,), 0)
    ones = jnp.ones((16,), jnp.int32)
    zeros16 = jnp.zeros((16,), jnp.int32)

    def row_body(r, carry):
        row = wid * ROWS_PER_W + r
        pltpu.sync_copy(logits.at[row], row_v)

        prefix = jnp.uint32(0)
        kk = jnp.int32(K)
        cgt = jnp.int32(0)
        for lvl in range(4):
            shift = jnp.uint32(24 - 8 * lvl)

            def clr(i, _):
                hist_v[pl.ds(i * 16, 16)] = zeros16
                return 0
            lax.fori_loop(0, 256, clr, 0)

            def hpass(i, _, prefix=prefix, shift=shift, lvl=lvl):
                x = row_v[pl.ds(i * 16, 16)]
                key = _key_of(x)
                bucket = (key >> shift) & jnp.uint32(255)
                idx = ((lane.astype(jnp.uint32) << jnp.uint32(8))
                       | bucket).astype(jnp.int32)
                if lvl == 0:
                    plsc.addupdate_scatter(hist_v, [idx], ones)
                else:
                    m = (key >> (shift + jnp.uint32(8))) == prefix
                    plsc.addupdate_scatter(hist_v, [idx], ones, mask=m)
                return 0
            lax.fori_loop(0, NV, hpass, 0)

            def chunk(j, c):
                total_above, found, boundary, above = c
                cb = 15 - j
                t = hist_v[pl.ds(cb * 16, 16)]
                for l in range(1, 16):
                    t = t + hist_v[pl.ds(l * 256 + cb * 16, 16)]
                rt = lax.rev(t, (0,))
                cs = plsc.cumsum(rt)
                cum = cs + total_above
                hit = cum >= kk
                nhit = _scalar(plsc.all_reduce_population_count(hit))
                pos = _scalar(plsc.all_reduce_ffs(hit))
                newly = (nhit > 0) & jnp.logical_not(found)
                prev = jnp.sum(jnp.where(lane == pos - 1, cs, 0))
                above_here = jnp.where(pos == 0, total_above,
                                       prev + total_above)
                boundary = jnp.where(newly, cb * 16 + 15 - pos, boundary)
                above = jnp.where(newly, above_here, above)
                found = found | (nhit > 0)
                total_above = total_above + jnp.sum(t)
                return total_above, found, boundary, above

            init = (jnp.int32(0), jnp.bool_(False), jnp.int32(0),
                    jnp.int32(0))
            _, _, boundary, above = lax.fori_loop(0, 16, chunk, init)
            prefix = (prefix << jnp.uint32(8)) | boundary.astype(jnp.uint32)
            kk = kk - above
            cgt = cgt + above

        key256 = prefix
        vb = key256 ^ jnp.where(key256 >= SIGN, SIGN, ALLON)
        v256 = plsc.bitcast(jnp.broadcast_to(vb, (16,)), jnp.float32)
        v256s = jnp.max(v256)

        def cpass(i, c):
            s_off, e_off = c
            x = row_v[pl.ds(i * 16, 16)]
            gidx = i * 16 + lane
            mgt = x > v256s
            meq = x == v256s
            plsc.store_compressed(stag_v.at[pl.ds(s_off, 16)], x, mask=mgt)
            plsc.store_compressed(stag_i.at[pl.ds(s_off, 16)], gidx, mask=mgt)
            eb = jnp.minimum(cgt + e_off, jnp.int32(272))
            plsc.store_compressed(stag_v.at[pl.ds(eb, 16)], x, mask=meq)
            plsc.store_compressed(stag_i.at[pl.ds(eb, 16)], gidx, mask=meq)
            s_off = s_off + _scalar(plsc.all_reduce_population_count(mgt))
            e_off = e_off + _scalar(plsc.all_reduce_population_count(meq))
            return s_off, e_off

        lax.fori_loop(0, NV, cpass, (jnp.int32(0), jnp.int32(0)))

        pltpu.sync_copy(stag_v.at[pl.ds(0, K)], out_v.at[row])
        pltpu.sync_copy(stag_i.at[pl.ds(0, K)], out_i.at[row])
        return carry

    lax.fori_loop(0, ROWS_PER_W, row_body, 0)


def _sc_select(padded):
    mesh = plsc.VectorSubcoreMesh(core_axis_name="c", subcore_axis_name="s")
    fn = pl.kernel(
        _sc_body,
        mesh=mesh,
        compiler_params=pltpu.CompilerParams(needs_layout_passes=False),
        out_type=[
            jax.ShapeDtypeStruct((BATCH, K), jnp.float32),
            jax.ShapeDtypeStruct((BATCH, K), jnp.int32),
        ],
        scratch_types=[
            pltpu.VMEM((VP,), jnp.float32),
            pltpu.VMEM((4096,), jnp.int32),
            pltpu.VMEM((STAG,), jnp.float32),
            pltpu.VMEM((STAG,), jnp.int32),
        ],
    )
    return fn(padded)


def _tc_body(cv_ref, ci_ref, params_ref, out_ref):
    v = cv_ref[...]
    ci = ci_ref[...].astype(jnp.float32)
    top_k = params_ref[:, 0].reshape(BATCH, 1)
    top_p = params_ref[:, 1].reshape(BATCH, 1)
    temperature = params_ref[:, 2].reshape(BATCH, 1)

    rank = jnp.zeros((BATCH, K), jnp.float32)
    for j in range(K):
        vj = v[:, j:j + 1]
        ij = ci[:, j:j + 1]
        gt = (vj > v) | ((vj == v) & (ij < ci))
        rank = rank + gt.astype(jnp.float32)

    lanes = lax.broadcasted_iota(jnp.int32, (BATCH, K), 1)
    lanesf = lanes.astype(jnp.float32)
    sorted_v = jnp.zeros((BATCH, K), jnp.float32)
    sorted_if = jnp.zeros((BATCH, K), jnp.float32)
    for j in range(K):
        rj = rank[:, j:j + 1]
        hit = (lanesf == rj).astype(jnp.float32)
        sorted_v = sorted_v + hit * v[:, j:j + 1]
        sorted_if = sorted_if + hit * ci[:, j:j + 1]

    pos = lanes.astype(jnp.float32)
    sl = jnp.where(pos >= top_k, IGNORED, sorted_v)
    sl = sl / temperature

    li = lax.broadcasted_iota(jnp.int32, (K, K), 0)
    lj = lax.broadcasted_iota(jnp.int32, (K, K), 1)
    tri = (li <= lj).astype(jnp.float32)

    def softmax_cumsum(x):
        m = jnp.max(x, axis=1, keepdims=True)
        e = jnp.exp(x - m)
        p = e / jnp.sum(e, axis=1, keepdims=True)
        return jnp.dot(p, tri, preferred_element_type=jnp.float32,
                       precision=lax.Precision.HIGHEST)

    cs = softmax_cumsum(sl)
    top_p_eff = jnp.maximum(jnp.min(cs), top_p)
    sl = jnp.where(cs > top_p_eff, IGNORED, sl)
    cs = softmax_cumsum(sl)

    counts = jnp.sum((0.5 > cs).astype(jnp.int32), axis=1, keepdims=True)
    picked = jnp.sum(jnp.where(lanes == counts, sorted_if, 0.0), axis=1,
                     keepdims=True)
    out_ref[...] = picked.astype(jnp.int32)


def _tc_sample(cand_v, cand_i, sampling_params):
    return pl.pallas_call(
        _tc_body,
        out_shape=jax.ShapeDtypeStruct((BATCH, 1), jnp.int32),
    )(cand_v, cand_i, sampling_params)


@jax.jit
def kernel(token_logits, sampling_params):
    cand_v, cand_i = _sc_select(token_logits)
    out = _tc_sample(cand_v, cand_i, sampling_params)
    return out.reshape(BATCH)


# speedup vs baseline: 3.3831x; 3.3831x over previous
import jax
import jax.numpy as jnp
import numpy as np
from jax import lax
from jax.experimental import pallas as pl
from jax.experimental.pallas import tpu as pltpu
from jax.experimental.pallas import tpu_sc as plsc

K = 256
IGNORED = -3000.0
PAD_VAL = -1e38

BATCH = 128
VOCAB = 100000
VP = VOCAB
NV = VP // 16
NWORK = 32
ROWS_PER_W = BATCH // NWORK
STAG = 304
SIGN = np.uint32(0x80000000)
ALLON = np.uint32(0xFFFFFFFF)


def _key_of(x):
    b = plsc.bitcast(x, jnp.uint32)
    neg = b >= SIGN
    return b ^ jnp.where(neg, ALLON, SIGN)


def _scalar(v):
    return jnp.max(v)


GROUP = 25
NGRP = NV // GROUP
CCAP = 4096


def _refine(read_vreg, n_loop, hist_v, lane, ones, zeros16, kk0):
    prefix = jnp.uint32(0)
    kk = kk0
    cgt = jnp.int32(0)
    for lvl in range(4):
        shift = jnp.uint32(24 - 8 * lvl)

        def clr(i, _):
            hist_v[pl.ds(i * 16, 16)] = zeros16
            return 0
        lax.fori_loop(0, 256, clr, 0)

        def hpass(i, _, prefix=prefix, shift=shift, lvl=lvl):
            x, valid = read_vreg(i)
            key = _key_of(x)
            bucket = (key >> shift) & jnp.uint32(255)
            idx = ((lane.astype(jnp.uint32) << jnp.uint32(8))
                   | bucket).astype(jnp.int32)
            if lvl == 0:
                m = valid
            else:
                m = (key >> (shift + jnp.uint32(8))) == prefix
                if valid is not None:
                    m = m & valid
            if m is None:
                plsc.addupdate_scatter(hist_v, [idx], ones)
            else:
                plsc.addupdate_scatter(hist_v, [idx], ones, mask=m)
            return 0
        lax.fori_loop(0, n_loop, hpass, 0)

        def chunk(j, c):
            total_above, found, boundary, above = c
            cb = 15 - j
            t = hist_v[pl.ds(cb * 16, 16)]
            for l in range(1, 16):
                t = t + hist_v[pl.ds(l * 256 + cb * 16, 16)]
            rt = lax.rev(t, (0,))
            cs = plsc.cumsum(rt)
            cum = cs + total_above
            hit = cum >= kk
            nhit = _scalar(plsc.all_reduce_population_count(hit))
            pos = _scalar(plsc.all_reduce_ffs(hit))
            newly = (nhit > 0) & jnp.logical_not(found)
            prev = jnp.sum(jnp.where(lane == pos - 1, cs, 0))
            above_here = jnp.where(pos == 0, total_above, prev + total_above)
            boundary = jnp.where(newly, cb * 16 + 15 - pos, boundary)
            above = jnp.where(newly, above_here, above)
            found = found | (nhit > 0)
            total_above = total_above + jnp.sum(t)
            return total_above, found, boundary, above

        init = (jnp.int32(0), jnp.bool_(False), jnp.int32(0), jnp.int32(0))
        _, _, boundary, above = lax.fori_loop(0, 16, chunk, init)
        prefix = (prefix << jnp.uint32(8)) | boundary.astype(jnp.uint32)
        kk = kk - above
        cgt = cgt + above
    return prefix, cgt


def _key_to_f32(key):
    vb = key ^ jnp.where(key >= SIGN, SIGN, ALLON)
    return jnp.max(plsc.bitcast(jnp.broadcast_to(vb, (16,)), jnp.float32))


def _sc_body(logits, out_v, out_i, row_v, maxima_v, hist_v, cand_v, cand_i,
             stag_v, stag_i):
    wid = lax.axis_index("s") * 2 + lax.axis_index("c")
    lane = lax.broadcasted_iota(jnp.int32, (16,), 0)
    ones = jnp.ones((16,), jnp.int32)
    zeros16 = jnp.zeros((16,), jnp.int32)

    def row_body(r, carry):
        row = wid * ROWS_PER_W + r
        pltpu.sync_copy(logits.at[row], row_v)

        def gmax(g, _):
            m = row_v[pl.ds(g * GROUP * 16, 16)]
            for u in range(1, GROUP):
                m = jnp.maximum(m, row_v[pl.ds((g * GROUP + u) * 16, 16)])
            maxima_v[pl.ds(g * 16, 16)] = m
            return 0
        lax.fori_loop(0, NGRP, gmax, 0)

        def read_max(i):
            return maxima_v[pl.ds(i * 16, 16)], None
        keyT, _ = _refine(read_max, NGRP, hist_v, lane, ones, zeros16,
                          jnp.int32(K))
        tf = _key_to_f32(keyT)

        def gpass(g, c_off):
            mx = maxima_v[pl.ds(g * 16, 16)]
            anyhit = _scalar(plsc.all_reduce_population_count(mx >= tf))

            def scan_group(c_off):
                def inner(u, c_off):
                    i = g * GROUP + u
                    x = row_v[pl.ds(i * 16, 16)]
                    gidx = i * 16 + lane
                    mk = x >= tf
                    off = jnp.minimum(c_off, jnp.int32(CCAP - 16))
                    plsc.store_compressed(cand_v.at[pl.ds(off, 16)], x,
                                          mask=mk)
                    plsc.store_compressed(cand_i.at[pl.ds(off, 16)], gidx,
                                          mask=mk)
                    return c_off + _scalar(
                        plsc.all_reduce_population_count(mk))
                return lax.fori_loop(0, GROUP, inner, c_off)

            return lax.cond(anyhit > 0, scan_group, lambda c: c, c_off)

        c_cnt = lax.fori_loop(0, NGRP, gpass, jnp.int32(0))
        ntrip = (c_cnt + 15) // 16

        def read_cand(i):
            x = cand_v[pl.ds(i * 16, 16)]
            return x, (i * 16 + lane) < c_cnt
        key256, cgt = _refine(read_cand, ntrip, hist_v, lane, ones, zeros16,
                              jnp.int32(K))
        v256s = _key_to_f32(key256)

        def cpass(i, c):
            s_off, e_off = c
            x = cand_v[pl.ds(i * 16, 16)]
            gidx = cand_i[pl.ds(i * 16, 16)]
            valid = (i * 16 + lane) < c_cnt
            mgt = (x > v256s) & valid
            meq = (x == v256s) & valid
            plsc.store_compressed(stag_v.at[pl.ds(s_off, 16)], x, mask=mgt)
            plsc.store_compressed(stag_i.at[pl.ds(s_off, 16)], gidx, mask=mgt)
            eb = jnp.minimum(cgt + e_off, jnp.int32(272))
            plsc.store_compressed(stag_v.at[pl.ds(eb, 16)], x, mask=meq)
            plsc.store_compressed(stag_i.at[pl.ds(eb, 16)], gidx, mask=meq)
            s_off = s_off + _scalar(plsc.all_reduce_population_count(mgt))
            e_off = e_off + _scalar(plsc.all_reduce_population_count(meq))
            return s_off, e_off

        lax.fori_loop(0, ntrip, cpass, (jnp.int32(0), jnp.int32(0)))

        pltpu.sync_copy(stag_v.at[pl.ds(0, K)], out_v.at[row])
        pltpu.sync_copy(stag_i.at[pl.ds(0, K)], out_i.at[row])
        return carry

    lax.fori_loop(0, ROWS_PER_W, row_body, 0)


def _sc_select(padded):
    mesh = plsc.VectorSubcoreMesh(core_axis_name="c", subcore_axis_name="s")
    fn = pl.kernel(
        _sc_body,
        mesh=mesh,
        compiler_params=pltpu.CompilerParams(needs_layout_passes=False),
        out_type=[
            jax.ShapeDtypeStruct((BATCH, K), jnp.float32),
            jax.ShapeDtypeStruct((BATCH, K), jnp.int32),
        ],
        scratch_types=[
            pltpu.VMEM((VP,), jnp.float32),
            pltpu.VMEM((NGRP * 16,), jnp.float32),
            pltpu.VMEM((4096,), jnp.int32),
            pltpu.VMEM((CCAP,), jnp.float32),
            pltpu.VMEM((CCAP,), jnp.int32),
            pltpu.VMEM((STAG,), jnp.float32),
            pltpu.VMEM((STAG,), jnp.int32),
        ],
    )
    return fn(padded)


def _tc_body(cv_ref, ci_ref, params_ref, out_ref):
    v = cv_ref[...]
    ci = ci_ref[...].astype(jnp.float32)
    top_k = params_ref[:, 0].reshape(BATCH, 1)
    top_p = params_ref[:, 1].reshape(BATCH, 1)
    temperature = params_ref[:, 2].reshape(BATCH, 1)

    rank = jnp.zeros((BATCH, K), jnp.float32)
    for j in range(K):
        vj = v[:, j:j + 1]
        ij = ci[:, j:j + 1]
        gt = (vj > v) | ((vj == v) & (ij < ci))
        rank = rank + gt.astype(jnp.float32)

    lanes = lax.broadcasted_iota(jnp.int32, (BATCH, K), 1)
    lanesf = lanes.astype(jnp.float32)
    sorted_v = jnp.zeros((BATCH, K), jnp.float32)
    sorted_if = jnp.zeros((BATCH, K), jnp.float32)
    for j in range(K):
        rj = rank[:, j:j + 1]
        hit = (lanesf == rj).astype(jnp.float32)
        sorted_v = sorted_v + hit * v[:, j:j + 1]
        sorted_if = sorted_if + hit * ci[:, j:j + 1]

    pos = lanes.astype(jnp.float32)
    sl = jnp.where(pos >= top_k, IGNORED, sorted_v)
    sl = sl / temperature

    li = lax.broadcasted_iota(jnp.int32, (K, K), 0)
    lj = lax.broadcasted_iota(jnp.int32, (K, K), 1)
    tri = (li <= lj).astype(jnp.float32)

    def softmax_cumsum(x):
        m = jnp.max(x, axis=1, keepdims=True)
        e = jnp.exp(x - m)
        p = e / jnp.sum(e, axis=1, keepdims=True)
        return jnp.dot(p, tri, preferred_element_type=jnp.float32,
                       precision=lax.Precision.HIGHEST)

    cs = softmax_cumsum(sl)
    top_p_eff = jnp.maximum(jnp.min(cs), top_p)
    sl = jnp.where(cs > top_p_eff, IGNORED, sl)
    cs = softmax_cumsum(sl)

    counts = jnp.sum((0.5 > cs).astype(jnp.int32), axis=1, keepdims=True)
    picked = jnp.sum(jnp.where(lanes == counts, sorted_if, 0.0), axis=1,
                     keepdims=True)
    out_ref[...] = picked.astype(jnp.int32)


def _tc_sample(cand_v, cand_i, sampling_params):
    return pl.pallas_call(
        _tc_body,
        out_shape=jax.ShapeDtypeStruct((BATCH, 1), jnp.int32),
    )(cand_v, cand_i, sampling_params)


@jax.jit
def kernel(token_logits, sampling_params):
    cand_v, cand_i = _sc_select(token_logits)
    out = _tc_sample(cand_v, cand_i, sampling_params)
    return out.reshape(BATCH)


# speedup vs baseline: 4.1519x; 1.2273x over previous
import jax
import jax.numpy as jnp
import numpy as np
from jax import lax
from jax.experimental import pallas as pl
from jax.experimental.pallas import tpu as pltpu
from jax.experimental.pallas import tpu_sc as plsc

K = 256
IGNORED = -3000.0
PAD_VAL = -1e38

BATCH = 128
VOCAB = 100000
VP = VOCAB
NV = VP // 16
NWORK = 32
ROWS_PER_W = BATCH // NWORK
STAG = 304
SIGN = np.uint32(0x80000000)
ALLON = np.uint32(0xFFFFFFFF)


def _key_of(x):
    b = plsc.bitcast(x, jnp.uint32)
    neg = b >= SIGN
    return b ^ jnp.where(neg, ALLON, SIGN)


def _scalar(v):
    return v[0]


GROUP = 25
NGRP = NV // GROUP
CCAP = 4096


def _refine(read_vreg, n_loop, hist_v, lane, ones, zeros16, kk0,
            levels=4, unroll=1):
    prefix = jnp.uint32(0)
    kk = kk0
    cgt = jnp.int32(0)
    for lvl in range(levels):
        shift = jnp.uint32(24 - 8 * lvl)

        def clr(i, _):
            hist_v[pl.ds(i * 16, 16)] = zeros16
            return 0
        lax.fori_loop(0, 256, clr, 0, unroll=8)

        def hpass(i, _, prefix=prefix, shift=shift, lvl=lvl):
            x, valid = read_vreg(i)
            key = _key_of(x)
            bucket = (key >> shift) & jnp.uint32(255)
            idx = ((lane.astype(jnp.uint32) << jnp.uint32(8))
                   | bucket).astype(jnp.int32)
            if lvl == 0:
                m = valid
            else:
                m = (key >> (shift + jnp.uint32(8))) == prefix
                if valid is not None:
                    m = m & valid
            if m is None:
                plsc.addupdate_scatter(hist_v, [idx], ones)
            else:
                plsc.addupdate_scatter(hist_v, [idx], ones, mask=m)
            return 0
        lax.fori_loop(0, n_loop, hpass, 0, unroll=unroll)

        def chunk(j, c):
            total_above, found, boundary, above = c
            cb = 15 - j
            t = hist_v[pl.ds(cb * 16, 16)]
            for l in range(1, 16):
                t = t + hist_v[pl.ds(l * 256 + cb * 16, 16)]
            rt = lax.rev(t, (0,))
            cs = plsc.cumsum(rt)
            cum = cs + total_above
            hit = cum >= kk
            nhit = _scalar(plsc.all_reduce_population_count(hit))
            pos = _scalar(plsc.all_reduce_ffs(hit))
            newly = (nhit > 0) & jnp.logical_not(found)
            prev = jnp.sum(jnp.where(lane == pos - 1, cs, 0))
            above_here = jnp.where(pos == 0, total_above, prev + total_above)
            boundary = jnp.where(newly, cb * 16 + 15 - pos, boundary)
            above = jnp.where(newly, above_here, above)
            found = found | (nhit > 0)
            total_above = total_above + jnp.sum(t)
            return total_above, found, boundary, above

        init = (jnp.int32(0), jnp.bool_(False), jnp.int32(0), jnp.int32(0))
        _, _, boundary, above = lax.fori_loop(0, 16, chunk, init)
        prefix = (prefix << jnp.uint32(8)) | boundary.astype(jnp.uint32)
        kk = kk - above
        cgt = cgt + above
    prefix = prefix << jnp.uint32(8 * (4 - levels))
    return prefix, cgt


def _key_to_f32(key):
    vb = key ^ jnp.where(key >= SIGN, SIGN, ALLON)
    return jnp.max(plsc.bitcast(jnp.broadcast_to(vb, (16,)), jnp.float32))


def _sc_body(logits, out_v, out_i, row_v, maxima_v, hist_v, cand_v, cand_i,
             stag_v, stag_i):
    wid = lax.axis_index("s") * 2 + lax.axis_index("c")
    lane = lax.broadcasted_iota(jnp.int32, (16,), 0)
    ones = jnp.ones((16,), jnp.int32)
    zeros16 = jnp.zeros((16,), jnp.int32)

    def row_body(r, carry):
        row = wid * ROWS_PER_W + r
        pltpu.sync_copy(logits.at[row], row_v)

        def gmax(g, _):
            m = row_v[pl.ds(g * GROUP * 16, 16)]
            for u in range(1, GROUP):
                m = jnp.maximum(m, row_v[pl.ds((g * GROUP + u) * 16, 16)])
            maxima_v[pl.ds(g * 16, 16)] = m
            return 0
        lax.fori_loop(0, NGRP, gmax, 0)

        def read_max(i):
            return maxima_v[pl.ds(i * 16, 16)], None
        keyT, _ = _refine(read_max, NGRP, hist_v, lane, ones, zeros16,
                          jnp.int32(K), levels=2, unroll=4)
        tf = _key_to_f32(keyT)

        def gpass(g, c_off):
            mx = maxima_v[pl.ds(g * 16, 16)]
            anyhit = _scalar(plsc.all_reduce_population_count(mx >= tf))

            def scan_group(c_off):
                def inner(u, c_off):
                    i = g * GROUP + u
                    x = row_v[pl.ds(i * 16, 16)]
                    gidx = i * 16 + lane
                    mk = x >= tf
                    off = jnp.minimum(c_off, jnp.int32(CCAP - 16))
                    plsc.store_compressed(cand_v.at[pl.ds(off, 16)], x,
                                          mask=mk)
                    plsc.store_compressed(cand_i.at[pl.ds(off, 16)], gidx,
                                          mask=mk)
                    return c_off + _scalar(
                        plsc.all_reduce_population_count(mk))
                return lax.fori_loop(0, GROUP, inner, c_off, unroll=4)

            return lax.cond(anyhit > 0, scan_group, lambda c: c, c_off)

        c_cnt = lax.fori_loop(0, NGRP, gpass, jnp.int32(0))
        c_cnt = jnp.minimum(c_cnt, jnp.int32(CCAP))
        ntrip = (c_cnt + 15) // 16

        def read_cand(i):
            x = cand_v[pl.ds(i * 16, 16)]
            return x, (i * 16 + lane) < c_cnt
        key256, cgt = _refine(read_cand, ntrip, hist_v, lane, ones, zeros16,
                              jnp.int32(K))
        v256s = _key_to_f32(key256)

        def cpass(i, c):
            s_off, e_off = c
            x = cand_v[pl.ds(i * 16, 16)]
            gidx = cand_i[pl.ds(i * 16, 16)]
            valid = (i * 16 + lane) < c_cnt
            mgt = (x > v256s) & valid
            meq = (x == v256s) & valid
            plsc.store_compressed(stag_v.at[pl.ds(s_off, 16)], x, mask=mgt)
            plsc.store_compressed(stag_i.at[pl.ds(s_off, 16)], gidx, mask=mgt)
            eb = jnp.minimum(cgt + e_off, jnp.int32(272))
            plsc.store_compressed(stag_v.at[pl.ds(eb, 16)], x, mask=meq)
            plsc.store_compressed(stag_i.at[pl.ds(eb, 16)], gidx, mask=meq)
            s_off = s_off + _scalar(plsc.all_reduce_population_count(mgt))
            e_off = e_off + _scalar(plsc.all_reduce_population_count(meq))
            return s_off, e_off

        lax.fori_loop(0, ntrip, cpass, (jnp.int32(0), jnp.int32(0)))

        pltpu.sync_copy(stag_v.at[pl.ds(0, K)], out_v.at[row])
        pltpu.sync_copy(stag_i.at[pl.ds(0, K)], out_i.at[row])
        return carry

    lax.fori_loop(0, ROWS_PER_W, row_body, 0)


def _sc_select(padded):
    mesh = plsc.VectorSubcoreMesh(core_axis_name="c", subcore_axis_name="s")
    fn = pl.kernel(
        _sc_body,
        mesh=mesh,
        compiler_params=pltpu.CompilerParams(needs_layout_passes=False),
        out_type=[
            jax.ShapeDtypeStruct((BATCH, K), jnp.float32),
            jax.ShapeDtypeStruct((BATCH, K), jnp.int32),
        ],
        scratch_types=[
            pltpu.VMEM((VP,), jnp.float32),
            pltpu.VMEM((NGRP * 16,), jnp.float32),
            pltpu.VMEM((4096,), jnp.int32),
            pltpu.VMEM((CCAP,), jnp.float32),
            pltpu.VMEM((CCAP,), jnp.int32),
            pltpu.VMEM((STAG,), jnp.float32),
            pltpu.VMEM((STAG,), jnp.int32),
        ],
    )
    return fn(padded)


def _tc_body(cv_ref, ci_ref, params_ref, out_ref):
    v = cv_ref[...]
    ci = ci_ref[...].astype(jnp.float32)
    top_k = params_ref[:, 0].reshape(BATCH, 1)
    top_p = params_ref[:, 1].reshape(BATCH, 1)
    temperature = params_ref[:, 2].reshape(BATCH, 1)

    racc = [jnp.zeros((BATCH, K), jnp.float32) for _ in range(4)]
    for j in range(K):
        vj = v[:, j:j + 1]
        ij = ci[:, j:j + 1]
        gt = (vj > v) | ((vj == v) & (ij < ci))
        racc[j % 4] = racc[j % 4] + gt.astype(jnp.float32)
    rank = (racc[0] + racc[1]) + (racc[2] + racc[3])

    lanes = lax.broadcasted_iota(jnp.int32, (BATCH, K), 1)
    lanesf = lanes.astype(jnp.float32)
    vacc = [jnp.zeros((BATCH, K), jnp.float32) for _ in range(4)]
    iacc = [jnp.zeros((BATCH, K), jnp.float32) for _ in range(4)]
    for j in range(K):
        rj = rank[:, j:j + 1]
        hit = (lanesf == rj).astype(jnp.float32)
        vacc[j % 4] = vacc[j % 4] + hit * v[:, j:j + 1]
        iacc[j % 4] = iacc[j % 4] + hit * ci[:, j:j + 1]
    sorted_v = (vacc[0] + vacc[1]) + (vacc[2] + vacc[3])
    sorted_if = (iacc[0] + iacc[1]) + (iacc[2] + iacc[3])

    pos = lanes.astype(jnp.float32)
    sl = jnp.where(pos >= top_k, IGNORED, sorted_v)
    sl = sl / temperature

    li = lax.broadcasted_iota(jnp.int32, (K, K), 0)
    lj = lax.broadcasted_iota(jnp.int32, (K, K), 1)
    tri = (li <= lj).astype(jnp.float32)

    def softmax_cumsum(x):
        m = jnp.max(x, axis=1, keepdims=True)
        e = jnp.exp(x - m)
        p = e / jnp.sum(e, axis=1, keepdims=True)
        return jnp.dot(p, tri, preferred_element_type=jnp.float32,
                       precision=lax.Precision.HIGHEST)

    cs = softmax_cumsum(sl)
    top_p_eff = jnp.maximum(jnp.min(cs), top_p)
    sl = jnp.where(cs > top_p_eff, IGNORED, sl)
    cs = softmax_cumsum(sl)

    counts = jnp.sum((0.5 > cs).astype(jnp.int32), axis=1, keepdims=True)
    picked = jnp.sum(jnp.where(lanes == counts, sorted_if, 0.0), axis=1,
                     keepdims=True)
    out_ref[...] = picked.astype(jnp.int32)


def _tc_sample(cand_v, cand_i, sampling_params):
    return pl.pallas_call(
        _tc_body,
        out_shape=jax.ShapeDtypeStruct((BATCH, 1), jnp.int32),
    )(cand_v, cand_i, sampling_params)


@jax.jit
def kernel(token_logits, sampling_params):
    cand_v, cand_i = _sc_select(token_logits)
    out = _tc_sample(cand_v, cand_i, sampling_params)
    return out.reshape(BATCH)
